# baseline (speedup 1.0000x reference)
import jax
import jax.numpy as jnp
from jax.experimental import pallas as pl


def _layernorm(x, g, b):
    mu = x.mean(-1, keepdims=True)
    var = x.var(-1, keepdims=True)
    return (x - mu) / jnp.sqrt(var + 1e-5) * g + b


def _cheb(x, src, dst, ew, Ws, b):
    n = x.shape[0]
    loop = jnp.arange(n)
    s = jnp.concatenate([src, loop])
    d = jnp.concatenate([dst, loop])
    w = jnp.concatenate([ew, jnp.ones((n,), x.dtype)])
    deg = jnp.zeros((n,), x.dtype).at[d].add(w)
    dinv = jnp.where(deg > 0, deg ** -0.5, 0.0)
    norm = -dinv[s] * w * dinv[d]

    def lhat(v):
        return jnp.zeros_like(v).at[d].add(norm[:, None] * v[s])

    Tx0 = x
    out = Tx0 @ Ws[0]
    if len(Ws) > 1:
        Tx1 = lhat(Tx0)
        out = out + Tx1 @ Ws[1]
        for k in range(2, len(Ws)):
            Tx2 = 2.0 * lhat(Tx1) - Tx0
            out = out + Tx2 @ Ws[k]
            Tx0, Tx1 = Tx1, Tx2
    return out + b


def _div_kernel(s_ref, c_ref, o_ref):
    o_ref[...] = s_ref[...] / jnp.clip(c_ref[...], 1.0)


def kernel(x, edge_attr, params, edge_index, batch):
    src = edge_index[0]
    dst = edge_index[1]
    G = 64
    for i in range(4):
        p = params['layer%d' % i]
        ew = jax.nn.sigmoid(edge_attr @ p['we'])
        x = _cheb(x, src, dst, ew, p['Ws'], p['b'])
        x = jax.nn.relu(_layernorm(x, p['gamma'], p['beta']))
    sums = jax.ops.segment_sum(x, batch, num_segments=G)
    cnt = jax.ops.segment_sum(jnp.ones((x.shape[0],), x.dtype), batch, num_segments=G)
    cnt2 = jnp.broadcast_to(cnt[:, None], sums.shape)
    return pl.pallas_call(
        _div_kernel,
        out_shape=jax.ShapeDtypeStruct(sums.shape, sums.dtype),
    )(sums, cnt2)


# speedup vs baseline: 13.9920x; 13.9920x over previous
import functools

import jax
import jax.numpy as jnp
from jax import lax
from jax.experimental import pallas as pl
from jax.experimental.pallas import tpu as pltpu
from jax.experimental.pallas import tpu_sc as plsc

N = 10000
E = 320000
D = 128
G = 64
NC = 2
NS = 16
NW = NC * NS
CH = 128
NCHUNK = 79
EPT = NCHUNK * CH
EPAD = NW * EPT
NPAD = 10240
ROWS_PT = NPAD // NS
RB = 128
ACCN = 32768
DEG_PT = ACCN // NS
NLAY = 3
LANES = 16


def _zero_rows(buf, nrows, width):
    def body(r, c):
        for j in range(width // LANES):
            buf.at[r][pl.ds(j * LANES, LANES)] = jnp.zeros((LANES,),
                                                           jnp.float32)
        return c
    lax.fori_loop(0, nrows, body, 0)


def _deg_body(dst_hbm, ew_hbm, out_hbm, dst_v, ew_v, idx_v, zbuf, acc_s):
    cid = lax.axis_index("c")
    sid = lax.axis_index("s")
    wid = sid * NC + cid

    pltpu.sync_copy(dst_hbm.at[wid], dst_v)
    for l in range(NLAY):
        pltpu.sync_copy(ew_hbm.at[l, wid], ew_v.at[l])

    def zb(r, c):
        zbuf[pl.ds(r * LANES, LANES)] = jnp.zeros((LANES,), jnp.float32)
        return c
    lax.fori_loop(0, DEG_PT // LANES, zb, 0)
    pltpu.sync_copy(zbuf, acc_s.at[pl.ds(sid * DEG_PT, DEG_PT)])
    plsc.subcore_barrier()

    def chunk(g, c):
        for l in range(NLAY):
            for j in range(CH // LANES):
                sl = pl.ds(j * LANES, LANES)
                idx_v.at[l][sl] = dst_v.at[g][sl] + l * N
            pltpu.sync_copy(ew_v.at[l, g], acc_s.at[idx_v.at[l]], add=True)
        return c

    lax.fori_loop(0, NCHUNK, chunk, 0)
    plsc.subcore_barrier()

    pltpu.sync_copy(acc_s.at[pl.ds(sid * DEG_PT, DEG_PT)], zbuf)
    pltpu.sync_copy(zbuf, out_hbm.at[cid, 0, pl.ds(sid * DEG_PT, DEG_PT)])


_deg_call = pl.kernel(
    _deg_body,
    out_type=jax.ShapeDtypeStruct((NC, 1, ACCN), jnp.float32),
    mesh=plsc.VectorSubcoreMesh(
        core_axis_name="c", subcore_axis_name="s", num_cores=NC,
        num_subcores=NS),
    scratch_types=[
        pltpu.VMEM((NCHUNK, CH), jnp.int32),
        pltpu.VMEM((NLAY, NCHUNK, CH), jnp.float32),
        pltpu.VMEM((NLAY, CH), jnp.int32),
        pltpu.VMEM((DEG_PT,), jnp.float32),
        pltpu.VMEM_SHARED((ACCN,), jnp.float32),
    ],
    compiler_params=pltpu.CompilerParams(needs_layout_passes=False),
)


def _lhat_body(v_hbm, src_hbm, dst_hbm, ew_hbm, dinv_hbm, out_hbm,
               src_v, dst_v, ewc_v, normc_v, dinv_v, rows_v, acc_s):
    cid = lax.axis_index("c")
    sid = lax.axis_index("s")
    wid = sid * NC + cid

    pltpu.sync_copy(src_hbm.at[wid], src_v)
    pltpu.sync_copy(dst_hbm.at[wid], dst_v)
    pltpu.sync_copy(dinv_hbm, dinv_v)

    _zero_rows(rows_v, RB, D)
    base = sid * ROWS_PT
    for k in range(ROWS_PT // RB):
        pltpu.sync_copy(rows_v, acc_s.at[pl.ds(base + k * RB, RB)])
    plsc.subcore_barrier()

    def chunk(g, c):
        pltpu.sync_copy(ew_hbm.at[wid, g, 0], ewc_v)
        pltpu.sync_copy(v_hbm.at[src_v.at[g]], rows_v)
        for j in range(CH // LANES):
            sl = pl.ds(j * LANES, LANES)
            s16 = src_v.at[g][sl]
            d16 = dst_v.at[g][sl]
            w16 = ewc_v[sl]
            dvs = plsc.load_gather(dinv_v, [s16])
            dvd = plsc.load_gather(dinv_v, [d16])
            normc_v[sl] = -(dvs * w16 * dvd)

        def srow(e, cc):
            ev = jnp.full((LANES,), e, jnp.int32)
            sc = plsc.load_gather(normc_v, [ev])
            for j in range(D // LANES):
                sl = pl.ds(j * LANES, LANES)
                rows_v.at[e][sl] = rows_v.at[e][sl] * sc
            return cc

        lax.fori_loop(0, CH, srow, 0)
        pltpu.sync_copy(rows_v, acc_s.at[dst_v.at[g]], add=True)
        return c

    lax.fori_loop(0, NCHUNK, chunk, 0)
    plsc.subcore_barrier()

    for k in range(ROWS_PT // RB):
        r0 = base + k * RB
        pltpu.sync_copy(acc_s.at[pl.ds(r0, RB)], rows_v)
        pltpu.sync_copy(rows_v, out_hbm.at[cid, pl.ds(r0, RB)])


_lhat_call = pl.kernel(
    _lhat_body,
    out_type=jax.ShapeDtypeStruct((NC, NPAD, D), jnp.float32),
    mesh=plsc.VectorSubcoreMesh(
        core_axis_name="c", subcore_axis_name="s", num_cores=NC,
        num_subcores=NS),
    scratch_types=[
        pltpu.VMEM((NCHUNK, CH), jnp.int32),
        pltpu.VMEM((NCHUNK, CH), jnp.int32),
        pltpu.VMEM((CH,), jnp.float32),
        pltpu.VMEM((CH,), jnp.float32),
        pltpu.VMEM((N,), jnp.float32),
        pltpu.VMEM((CH, D), jnp.float32),
        pltpu.VMEM_SHARED((NPAD, D), jnp.float32),
    ],
    compiler_params=pltpu.CompilerParams(needs_layout_passes=False),
)


def _ew_body(we_ref, ea_ref, o_ref):
    z = lax.dot_general(we_ref[...], ea_ref[...], (((1,), (1,)), ((), ())),
                        preferred_element_type=jnp.float32)
    o_ref[...] = 1.0 / (1.0 + jnp.exp(-z))


def _ew_sigmoid(we_t, edge_attr):
    eb = 16000
    return pl.pallas_call(
        _ew_body,
        grid=(E // eb,),
        in_specs=[
            pl.BlockSpec((NLAY, 16), lambda i: (0, 0)),
            pl.BlockSpec((eb, 16), lambda i: (i, 0)),
        ],
        out_specs=pl.BlockSpec((NLAY, eb), lambda i: (0, i)),
        out_shape=jax.ShapeDtypeStruct((NLAY, E), jnp.float32),
    )(we_t, edge_attr)


def _dinv_body(degp_ref, dinv_ref, dinv2_ref):
    dtot = degp_ref[0] + degp_ref[1] + 1.0
    dinv2 = 1.0 / dtot
    dinv_ref[...] = lax.rsqrt(dtot)
    dinv2_ref[...] = dinv2


def _dinv_fn(degp):
    return pl.pallas_call(
        _dinv_body,
        out_shape=(
            jax.ShapeDtypeStruct((NLAY, N), jnp.float32),
            jax.ShapeDtypeStruct((NLAY, N), jnp.float32),
        ),
    )(degp)


def _ln_relu(h, g, b):
    mu = jnp.mean(h, axis=-1, keepdims=True)
    var = jnp.mean((h - mu) * (h - mu), axis=-1, keepdims=True)
    hn = (h - mu) * lax.rsqrt(var + 1e-5) * g + b
    return jnp.maximum(hn, 0.0)


_RB_TC = 2000
_GRID_TC = N // _RB_TC


def _rows_spec():
    return pl.BlockSpec((_RB_TC, D), lambda i: (i, 0))


def _part_spec():
    return pl.BlockSpec((NC, _RB_TC, D), lambda i: (0, i, 0))


def _vec_spec():
    return pl.BlockSpec((1, D), lambda i: (0, 0))


def _col_spec():
    return pl.BlockSpec((_RB_TC, 1), lambda i: (i, 0))


def _w_spec():
    return pl.BlockSpec((D, D), lambda i: (0, 0))


def _dense1_body(x_ref, w0_ref, b_ref, g_ref, be_ref, o_ref):
    h = jnp.dot(x_ref[...], w0_ref[...], preferred_element_type=jnp.float32)
    o_ref[...] = _ln_relu(h + b_ref[...], g_ref[...], be_ref[...])


def _dense1(x, w0, b, g, be):
    return pl.pallas_call(
        _dense1_body,
        grid=(_GRID_TC,),
        in_specs=[_rows_spec(), _w_spec(), _vec_spec(), _vec_spec(),
                  _vec_spec()],
        out_specs=_rows_spec(),
        out_shape=jax.ShapeDtypeStruct((N, D), jnp.float32),
    )(x, w0, b, g, be)


def _dense2_body(x_ref, p_ref, d2_ref, w0_ref, w1_ref, b_ref, g_ref, be_ref,
                 o_ref):
    x = x_ref[...]
    t1 = p_ref[0] + p_ref[1] - d2_ref[...] * x
    h = (jnp.dot(x, w0_ref[...], preferred_element_type=jnp.float32)
         + jnp.dot(t1, w1_ref[...], preferred_element_type=jnp.float32)
         + b_ref[...])
    o_ref[...] = _ln_relu(h, g_ref[...], be_ref[...])


def _dense2(x, p, d2, w0, w1, b, g, be):
    return pl.pallas_call(
        _dense2_body,
        grid=(_GRID_TC,),
        in_specs=[_rows_spec(), _part_spec(), _col_spec(), _w_spec(),
                  _w_spec(), _vec_spec(), _vec_spec(), _vec_spec()],
        out_specs=_rows_spec(),
        out_shape=jax.ShapeDtypeStruct((N, D), jnp.float32),
    )(x, p, d2, w0, w1, b, g, be)


def _comb_body(x_ref, p_ref, d2_ref, o_ref):
    o_ref[...] = p_ref[0] + p_ref[1] - d2_ref[...] * x_ref[...]


def _comb(x, p, d2):
    return pl.pallas_call(
        _comb_body,
        grid=(_GRID_TC,),
        in_specs=[_rows_spec(), _part_spec(), _col_spec()],
        out_specs=_rows_spec(),
        out_shape=jax.ShapeDtypeStruct((N, D), jnp.float32),
    )(x, p, d2)


def _dense3_body(x_ref, t1_ref, q_ref, d2_ref, w0_ref, w1_ref, w2_ref, b_ref,
                 g_ref, be_ref, o_ref):
    x = x_ref[...]
    t1 = t1_ref[...]
    t2 = 2.0 * (q_ref[0] + q_ref[1] - d2_ref[...] * t1) - x
    h = (jnp.dot(x, w0_ref[...], preferred_element_type=jnp.float32)
         + jnp.dot(t1, w1_ref[...], preferred_element_type=jnp.float32)
         + jnp.dot(t2, w2_ref[...], preferred_element_type=jnp.float32)
         + b_ref[...])
    o_ref[...] = _ln_relu(h, g_ref[...], be_ref[...])


def _dense3(x, t1, q, d2, w0, w1, w2, b, g, be):
    return pl.pallas_call(
        _dense3_body,
        grid=(_GRID_TC,),
        in_specs=[_rows_spec(), _rows_spec(), _part_spec(), _col_spec(),
                  _w_spec(), _w_spec(), _w_spec(), _vec_spec(), _vec_spec(),
                  _vec_spec()],
        out_specs=_rows_spec(),
        out_shape=jax.ShapeDtypeStruct((N, D), jnp.float32),
    )(x, t1, q, d2, w0, w1, w2, b, g, be)


def _pool_body(x_ref, b_ref, o_ref, sum_s, cnt_s):
    i = pl.program_id(0)

    @pl.when(i == 0)
    def _():
        sum_s[...] = jnp.zeros((G, D), jnp.float32)
        cnt_s[...] = jnp.zeros((G, D), jnp.float32)

    bb = b_ref[0, 0, :]
    seg = lax.broadcasted_iota(jnp.int32, (G, _RB_TC), 0)
    oh = (seg == bb[None, :]).astype(jnp.float32)
    x = x_ref[...]
    sum_s[...] += jnp.dot(oh, x, preferred_element_type=jnp.float32)
    cnt_s[...] += jnp.dot(oh, jnp.ones_like(x),
                          preferred_element_type=jnp.float32)

    @pl.when(i == pl.num_programs(0) - 1)
    def _():
        o_ref[...] = sum_s[...] / jnp.maximum(cnt_s[...], 1.0)


def _pool(x, batch3):
    return pl.pallas_call(
        _pool_body,
        grid=(_GRID_TC,),
        in_specs=[
            _rows_spec(),
            pl.BlockSpec((1, 1, _RB_TC), lambda i: (i, 0, 0)),
        ],
        out_specs=pl.BlockSpec((G, D), lambda i: (0, 0)),
        out_shape=jax.ShapeDtypeStruct((G, D), jnp.float32),
        scratch_shapes=[
            pltpu.VMEM((G, D), jnp.float32),
            pltpu.VMEM((G, D), jnp.float32),
        ],
    )(x, batch3)


def kernel(x, edge_attr, params, edge_index, batch):
    src = edge_index[0].astype(jnp.int32)
    dst = edge_index[1].astype(jnp.int32)

    npad = EPAD - E
    padidx = jnp.arange(npad, dtype=jnp.int32)
    src3 = jnp.concatenate([src, padidx]).reshape(NW, NCHUNK, CH)
    dst3 = jnp.concatenate([dst, padidx]).reshape(NW, NCHUNK, CH)

    we_t = jnp.stack([params['layer%d' % i]['we'] for i in (1, 2, 3)])
    ew = _ew_sigmoid(we_t, edge_attr)
    ew3 = jnp.pad(ew, ((0, 0), (0, npad))).reshape(NLAY, NW, NCHUNK, CH)

    ew4 = ew3.reshape(NLAY, NW, NCHUNK, 1, CH)
    degp = _deg_call(dst3, ew3)
    degp = degp.reshape(NC, ACCN)[:, :NLAY * N].reshape(NC, NLAY, N)
    dinv3, dinv23 = _dinv_fn(degp)

    p = [params['layer%d' % i] for i in range(4)]

    def vec(a):
        return a.reshape(1, D)

    x1 = _dense1(x, p[0]['Ws'][0], vec(p[0]['b']), vec(p[0]['gamma']),
                 vec(p[0]['beta']))

    p1 = _lhat_call(x1, src3, dst3, ew4[0], dinv3[0])
    x2 = _dense2(x1, p1, dinv23[0].reshape(N, 1), p[1]['Ws'][0],
                 p[1]['Ws'][1], vec(p[1]['b']), vec(p[1]['gamma']),
                 vec(p[1]['beta']))

    p2 = _lhat_call(x2, src3, dst3, ew4[1], dinv3[1])
    x3 = _dense2(x2, p2, dinv23[1].reshape(N, 1), p[2]['Ws'][0],
                 p[2]['Ws'][1], vec(p[2]['b']), vec(p[2]['gamma']),
                 vec(p[2]['beta']))

    q1 = _lhat_call(x3, src3, dst3, ew4[2], dinv3[2])
    t1 = _comb(x3, q1, dinv23[2].reshape(N, 1))
    q2 = _lhat_call(t1, src3, dst3, ew4[2], dinv3[2])
    x4 = _dense3(x3, t1, q2, dinv23[2].reshape(N, 1), p[3]['Ws'][0],
                 p[3]['Ws'][1], p[3]['Ws'][2], vec(p[3]['b']),
                 vec(p[3]['gamma']), vec(p[3]['beta']))

    batch3 = batch.astype(jnp.int32).reshape(_GRID_TC, 1, _RB_TC)
    return _pool(x4, batch3)


# speedup vs baseline: 24.8831x; 1.7784x over previous
import jax
import jax.numpy as jnp
from jax import lax
from jax.experimental import pallas as pl
from jax.experimental.pallas import tpu as pltpu
from jax.experimental.pallas import tpu_sc as plsc

N = 10000
E = 320000
D = 128
G = 64
NC = 2
NS = 16
NW = NC * NS
CH = 112
NCHUNK = 90
EPT = NCHUNK * CH
EPAD = NW * EPT
RPT = 632
RPT_LAST = N - 15 * RPT
ACCN = 32768
DEG_PT = ACCN // NS
NLAY = 3
LANES = 16
NBUF = 3
NSLOT = 6
UNROLL = 6
NGRP = NCHUNK // UNROLL


def _zero_rows(buf, nrows, width):
    def body(r, c):
        for j in range(width // LANES):
            buf.at[r][pl.ds(j * LANES, LANES)] = jnp.zeros((LANES,),
                                                           jnp.float32)
        return c
    lax.fori_loop(0, nrows, body, 0)


def _deg_body(dst_hbm, ew_hbm, out_hbm, dst_v, ew_v, idx_v, zbuf, acc_s):
    cid = lax.axis_index("c")
    sid = lax.axis_index("s")
    wid = sid * NC + cid

    pltpu.sync_copy(dst_hbm.at[wid], dst_v)
    for l in range(NLAY):
        pltpu.sync_copy(ew_hbm.at[l, wid], ew_v.at[l])

    def zb(r, c):
        zbuf[pl.ds(r * LANES, LANES)] = jnp.zeros((LANES,), jnp.float32)
        return c
    lax.fori_loop(0, DEG_PT // LANES, zb, 0)
    pltpu.sync_copy(zbuf, acc_s.at[pl.ds(sid * DEG_PT, DEG_PT)])
    plsc.subcore_barrier()

    def chunk(g, c):
        for l in range(NLAY):
            for j in range(CH // LANES):
                sl = pl.ds(j * LANES, LANES)
                idx_v.at[l][sl] = dst_v.at[g][sl] + l * N
            pltpu.sync_copy(ew_v.at[l, g], acc_s.at[idx_v.at[l]], add=True)
        return c

    lax.fori_loop(0, NCHUNK, chunk, 0)
    plsc.subcore_barrier()

    pltpu.sync_copy(acc_s.at[pl.ds(sid * DEG_PT, DEG_PT)], zbuf)
    pltpu.sync_copy(zbuf, out_hbm.at[cid, 0, pl.ds(sid * DEG_PT, DEG_PT)])


_deg_call = pl.kernel(
    _deg_body,
    out_type=jax.ShapeDtypeStruct((NC, 1, ACCN), jnp.float32),
    mesh=plsc.VectorSubcoreMesh(
        core_axis_name="c", subcore_axis_name="s", num_cores=NC,
        num_subcores=NS),
    scratch_types=[
        pltpu.VMEM((NCHUNK, CH), jnp.int32),
        pltpu.VMEM((NLAY, NCHUNK, CH), jnp.float32),
        pltpu.VMEM((NLAY, CH), jnp.int32),
        pltpu.VMEM((DEG_PT,), jnp.float32),
        pltpu.VMEM_SHARED((ACCN,), jnp.float32),
    ],
    compiler_params=pltpu.CompilerParams(needs_layout_passes=False),
)


def _norm_body(sd_hbm, ew_hbm, dinv_hbm, out_hbm, sd_v, ew_v, dinv_v, o_v):
    cid = lax.axis_index("c")
    sid = lax.axis_index("s")
    wid = sid * NC + cid

    pltpu.sync_copy(sd_hbm.at[wid], sd_v)
    pltpu.sync_copy(dinv_hbm, dinv_v)

    def cp(g, c):
        for r in range(2):
            for j in range(CH // LANES):
                sl = pl.ds(j * LANES, LANES)
                o_v.at[g, r][sl] = sd_v.at[g, r][sl]
        return c

    lax.fori_loop(0, NCHUNK, cp, 0)

    for l in range(NLAY):
        pltpu.sync_copy(ew_hbm.at[l, wid], ew_v)
        l16 = jnp.full((LANES,), l, jnp.int32)

        def f(g, c):
            for j in range(CH // LANES):
                sl = pl.ds(j * LANES, LANES)
                s16 = sd_v.at[g, 0][sl]
                d16 = sd_v.at[g, 1][sl]
                w16 = ew_v.at[g][sl]
                dvs = plsc.load_gather(dinv_v, [l16, s16])
                dvd = plsc.load_gather(dinv_v, [l16, d16])
                o_v.at[g, 2][sl] = plsc.bitcast(-(dvs * w16 * dvd),
                                                jnp.int32)
            return c

        lax.fori_loop(0, NCHUNK, f, 0)
        pltpu.sync_copy(o_v, out_hbm.at[l, wid])


_norm_call = pl.kernel(
    _norm_body,
    out_type=jax.ShapeDtypeStruct((NLAY, NW, NCHUNK, 3, CH), jnp.int32),
    mesh=plsc.VectorSubcoreMesh(
        core_axis_name="c", subcore_axis_name="s", num_cores=NC,
        num_subcores=NS),
    scratch_types=[
        pltpu.VMEM((NCHUNK, 2, CH), jnp.int32),
        pltpu.VMEM((NCHUNK, CH), jnp.float32),
        pltpu.VMEM((NLAY, N), jnp.float32),
        pltpu.VMEM((NCHUNK, 3, CH), jnp.int32),
    ],
    compiler_params=pltpu.CompilerParams(needs_layout_passes=False),
)


def _lhat_body(v_hbm, ed_hbm, out_hbm, ering, rows, normc, acc_s,
               gsem, ssem, esem):
    cid = lax.axis_index("c")
    sid = lax.axis_index("s")
    wid = sid * NC + cid

    _zero_rows(rows.at[0], CH, D)
    base = sid * RPT

    @pl.when(sid < 15)
    def _():
        for k in range(RPT // CH):
            pltpu.sync_copy(rows.at[0], acc_s.at[pl.ds(base + k * CH, CH)])
        rem = RPT - (RPT // CH) * CH
        pltpu.sync_copy(rows.at[0, pl.ds(0, rem)],
                        acc_s.at[pl.ds(base + RPT - rem, rem)])

    @pl.when(sid == 15)
    def _():
        for k in range(RPT_LAST // CH):
            pltpu.sync_copy(rows.at[0], acc_s.at[pl.ds(base + k * CH, CH)])
        rem = RPT_LAST - (RPT_LAST // CH) * CH
        pltpu.sync_copy(rows.at[0, pl.ds(0, rem)],
                        acc_s.at[pl.ds(base + RPT_LAST - rem, rem)])

    plsc.subcore_barrier()

    def fetch_ed(c, slot):
        pltpu.async_copy(ed_hbm.at[wid, c], ering.at[slot], esem.at[slot])

    def wait_ed(slot):
        pltpu.make_async_copy(ed_hbm.at[wid, 0], ering.at[0],
                              esem.at[slot]).wait()

    def issue_gather(c, slot, b):
        pltpu.async_copy(v_hbm.at[ering.at[slot, 0]], rows.at[b], gsem.at[b])

    def wait_gather(b):
        pltpu.make_async_copy(v_hbm.at[pl.ds(0, CH)], rows.at[b],
                              gsem.at[b]).wait()

    def issue_scatter(slot, b):
        pltpu.async_copy(rows.at[b], acc_s.at[ering.at[slot, 1]],
                         ssem.at[b], add=True)

    def wait_scatter(b):
        pltpu.make_async_copy(rows.at[0], acc_s.at[pl.ds(0, CH)],
                              ssem.at[b]).wait()

    def scale(cdyn, slot, b):
        for j in range(CH // LANES):
            sl = pl.ds(j * LANES, LANES)
            normc[sl] = plsc.bitcast(ering.at[slot, 2][sl], jnp.float32)

        def srow(e, cc):
            ev = jnp.full((LANES,), e, jnp.int32)
            sc = plsc.load_gather(normc, [ev])
            for j in range(D // LANES):
                sl = pl.ds(j * LANES, LANES)
                rows.at[b, e][sl] = rows.at[b, e][sl] * sc
            return cc

        lax.fori_loop(0, CH, srow, 0, unroll=2)

    def step(c, k, first=False, pf=True, gi=True):
        b = k % NBUF
        wait_gather(b)
        scale(c, k % NSLOT, b)
        issue_scatter(k % NSLOT, b)
        if not first:
            wait_scatter((b + 2) % NBUF)
        if pf:
            fetch_ed(c + 4, (k + 4) % NSLOT)
        if gi:
            s2 = (k + 2) % NSLOT
            wait_ed(s2)
            issue_gather(c + 2, s2, (b + 2) % NBUF)

    for t in range(4):
        fetch_ed(t, t)
    for t in range(2):
        wait_ed(t)
        issue_gather(t, t, t)

    for k in range(UNROLL):
        step(k, k, first=(k == 0))

    def grp(g, c):
        c0 = g * UNROLL
        for k in range(UNROLL):
            step(c0 + k, k)
        return c

    lax.fori_loop(1, NGRP - 1, grp, 0)

    for k in range(UNROLL):
        c = NCHUNK - UNROLL + k
        step(c, k, pf=(c + 4 < NCHUNK), gi=(c + 2 < NCHUNK))

    wait_scatter((NCHUNK - 1) % NBUF)
    plsc.subcore_barrier()

    @pl.when(sid < 15)
    def _():
        pltpu.sync_copy(acc_s.at[pl.ds(base, RPT)],
                        out_hbm.at[cid, pl.ds(base, RPT)])

    @pl.when(sid == 15)
    def _():
        pltpu.sync_copy(acc_s.at[pl.ds(base, RPT_LAST)],
                        out_hbm.at[cid, pl.ds(base, RPT_LAST)])


_lhat_call = pl.kernel(
    _lhat_body,
    out_type=jax.ShapeDtypeStruct((NC, N, D), jnp.float32),
    mesh=plsc.VectorSubcoreMesh(
        core_axis_name="c", subcore_axis_name="s", num_cores=NC,
        num_subcores=NS),
    scratch_types=[
        pltpu.VMEM((NSLOT, 3, CH), jnp.int32),
        pltpu.VMEM((NBUF, CH, D), jnp.float32),
        pltpu.VMEM((CH,), jnp.float32),
        pltpu.VMEM_SHARED((N, D), jnp.float32),
        pltpu.SemaphoreType.DMA((NBUF,)),
        pltpu.SemaphoreType.DMA((NBUF,)),
        pltpu.SemaphoreType.DMA((NSLOT,)),
    ],
    compiler_params=pltpu.CompilerParams(needs_layout_passes=False),
)


def _ew_body(we_ref, ea_ref, o_ref):
    z = lax.dot_general(we_ref[...], ea_ref[...], (((1,), (1,)), ((), ())),
                        preferred_element_type=jnp.float32)
    o_ref[...] = 1.0 / (1.0 + jnp.exp(-z))


def _ew_sigmoid(we_t, edge_attr):
    eb = 16000
    return pl.pallas_call(
        _ew_body,
        grid=(E // eb,),
        in_specs=[
            pl.BlockSpec((NLAY, 16), lambda i: (0, 0)),
            pl.BlockSpec((eb, 16), lambda i: (i, 0)),
        ],
        out_specs=pl.BlockSpec((NLAY, eb), lambda i: (0, i)),
        out_shape=jax.ShapeDtypeStruct((NLAY, E), jnp.float32),
    )(we_t, edge_attr)


def _dinv_body(degp_ref, dinv_ref, dinv2_ref):
    dtot = degp_ref[0] + degp_ref[1] + 1.0
    dinv2 = 1.0 / dtot
    dinv_ref[...] = lax.rsqrt(dtot)
    dinv2_ref[...] = dinv2


def _dinv_fn(degp):
    return pl.pallas_call(
        _dinv_body,
        out_shape=(
            jax.ShapeDtypeStruct((NLAY, N), jnp.float32),
            jax.ShapeDtypeStruct((NLAY, N), jnp.float32),
        ),
    )(degp)


def _ln_relu(h, g, b):
    mu = jnp.mean(h, axis=-1, keepdims=True)
    var = jnp.mean((h - mu) * (h - mu), axis=-1, keepdims=True)
    hn = (h - mu) * lax.rsqrt(var + 1e-5) * g + b
    return jnp.maximum(hn, 0.0)


_RB_TC = 2000
_GRID_TC = N // _RB_TC


def _rows_spec():
    return pl.BlockSpec((_RB_TC, D), lambda i: (i, 0))


def _part_spec():
    return pl.BlockSpec((NC, _RB_TC, D), lambda i: (0, i, 0))


def _vec_spec():
    return pl.BlockSpec((1, D), lambda i: (0, 0))


def _col_spec():
    return pl.BlockSpec((_RB_TC, 1), lambda i: (i, 0))


def _w_spec():
    return pl.BlockSpec((D, D), lambda i: (0, 0))


def _dense1_body(x_ref, w0_ref, b_ref, g_ref, be_ref, o_ref):
    h = jnp.dot(x_ref[...], w0_ref[...], preferred_element_type=jnp.float32)
    o_ref[...] = _ln_relu(h + b_ref[...], g_ref[...], be_ref[...])


def _dense1(x, w0, b, g, be):
    return pl.pallas_call(
        _dense1_body,
        grid=(_GRID_TC,),
        in_specs=[_rows_spec(), _w_spec(), _vec_spec(), _vec_spec(),
                  _vec_spec()],
        out_specs=_rows_spec(),
        out_shape=jax.ShapeDtypeStruct((N, D), jnp.float32),
    )(x, w0, b, g, be)


def _dense2_body(x_ref, p_ref, d2_ref, w0_ref, w1_ref, b_ref, g_ref, be_ref,
                 o_ref):
    x = x_ref[...]
    t1 = p_ref[0] + p_ref[1] - d2_ref[...] * x
    h = (jnp.dot(x, w0_ref[...], preferred_element_type=jnp.float32)
         + jnp.dot(t1, w1_ref[...], preferred_element_type=jnp.float32)
         + b_ref[...])
    o_ref[...] = _ln_relu(h, g_ref[...], be_ref[...])


def _dense2(x, p, d2, w0, w1, b, g, be):
    return pl.pallas_call(
        _dense2_body,
        grid=(_GRID_TC,),
        in_specs=[_rows_spec(), _part_spec(), _col_spec(), _w_spec(),
                  _w_spec(), _vec_spec(), _vec_spec(), _vec_spec()],
        out_specs=_rows_spec(),
        out_shape=jax.ShapeDtypeStruct((N, D), jnp.float32),
    )(x, p, d2, w0, w1, b, g, be)


def _comb_body(x_ref, p_ref, d2_ref, o_ref):
    o_ref[...] = p_ref[0] + p_ref[1] - d2_ref[...] * x_ref[...]


def _comb(x, p, d2):
    return pl.pallas_call(
        _comb_body,
        grid=(_GRID_TC,),
        in_specs=[_rows_spec(), _part_spec(), _col_spec()],
        out_specs=_rows_spec(),
        out_shape=jax.ShapeDtypeStruct((N, D), jnp.float32),
    )(x, p, d2)


def _dense3_body(x_ref, t1_ref, q_ref, d2_ref, w0_ref, w1_ref, w2_ref, b_ref,
                 g_ref, be_ref, o_ref):
    x = x_ref[...]
    t1 = t1_ref[...]
    t2 = 2.0 * (q_ref[0] + q_ref[1] - d2_ref[...] * t1) - x
    h = (jnp.dot(x, w0_ref[...], preferred_element_type=jnp.float32)
         + jnp.dot(t1, w1_ref[...], preferred_element_type=jnp.float32)
         + jnp.dot(t2, w2_ref[...], preferred_element_type=jnp.float32)
         + b_ref[...])
    o_ref[...] = _ln_relu(h, g_ref[...], be_ref[...])


def _dense3(x, t1, q, d2, w0, w1, w2, b, g, be):
    return pl.pallas_call(
        _dense3_body,
        grid=(_GRID_TC,),
        in_specs=[_rows_spec(), _rows_spec(), _part_spec(), _col_spec(),
                  _w_spec(), _w_spec(), _w_spec(), _vec_spec(), _vec_spec(),
                  _vec_spec()],
        out_specs=_rows_spec(),
        out_shape=jax.ShapeDtypeStruct((N, D), jnp.float32),
    )(x, t1, q, d2, w0, w1, w2, b, g, be)


def _pool_body(x_ref, b_ref, o_ref, sum_s, cnt_s):
    i = pl.program_id(0)

    @pl.when(i == 0)
    def _():
        sum_s[...] = jnp.zeros((G, D), jnp.float32)
        cnt_s[...] = jnp.zeros((G, D), jnp.float32)

    bb = b_ref[0, 0, :]
    seg = lax.broadcasted_iota(jnp.int32, (G, _RB_TC), 0)
    oh = (seg == bb[None, :]).astype(jnp.float32)
    x = x_ref[...]
    sum_s[...] += jnp.dot(oh, x, preferred_element_type=jnp.float32)
    cnt_s[...] += jnp.dot(oh, jnp.ones_like(x),
                          preferred_element_type=jnp.float32)

    @pl.when(i == pl.num_programs(0) - 1)
    def _():
        o_ref[...] = sum_s[...] / jnp.maximum(cnt_s[...], 1.0)


def _pool(x, batch3):
    return pl.pallas_call(
        _pool_body,
        grid=(_GRID_TC,),
        in_specs=[
            _rows_spec(),
            pl.BlockSpec((1, 1, _RB_TC), lambda i: (i, 0, 0)),
        ],
        out_specs=pl.BlockSpec((G, D), lambda i: (0, 0)),
        out_shape=jax.ShapeDtypeStruct((G, D), jnp.float32),
        scratch_shapes=[
            pltpu.VMEM((G, D), jnp.float32),
            pltpu.VMEM((G, D), jnp.float32),
        ],
    )(x, batch3)


def kernel(x, edge_attr, params, edge_index, batch):
    src = edge_index[0].astype(jnp.int32)
    dst = edge_index[1].astype(jnp.int32)

    npad = EPAD - E
    padidx = jnp.arange(npad, dtype=jnp.int32)
    src_p = jnp.concatenate([src, padidx]).reshape(NW, NCHUNK, CH)
    dst_p = jnp.concatenate([dst, padidx]).reshape(NW, NCHUNK, CH)
    sd3 = jnp.stack([src_p, dst_p], axis=2)

    we_t = jnp.stack([params['layer%d' % i]['we'] for i in (1, 2, 3)])
    ew = _ew_sigmoid(we_t, edge_attr)
    ew3 = jnp.pad(ew, ((0, 0), (0, npad))).reshape(NLAY, NW, NCHUNK, CH)

    degp = _deg_call(dst_p, ew3)
    degp = degp.reshape(NC, ACCN)[:, :NLAY * N].reshape(NC, NLAY, N)
    dinv3, dinv23 = _dinv_fn(degp)
    ed3 = _norm_call(sd3, ew3, dinv3)

    p = [params['layer%d' % i] for i in range(4)]

    def vec(a):
        return a.reshape(1, D)

    x1 = _dense1(x, p[0]['Ws'][0], vec(p[0]['b']), vec(p[0]['gamma']),
                 vec(p[0]['beta']))

    p1 = _lhat_call(x1, ed3[0])
    x2 = _dense2(x1, p1, dinv23[0].reshape(N, 1), p[1]['Ws'][0],
                 p[1]['Ws'][1], vec(p[1]['b']), vec(p[1]['gamma']),
                 vec(p[1]['beta']))

    p2 = _lhat_call(x2, ed3[1])
    x3 = _dense2(x2, p2, dinv23[1].reshape(N, 1), p[2]['Ws'][0],
                 p[2]['Ws'][1], vec(p[2]['b']), vec(p[2]['gamma']),
                 vec(p[2]['beta']))

    q1 = _lhat_call(x3, ed3[2])
    t1 = _comb(x3, q1, dinv23[2].reshape(N, 1))
    q2 = _lhat_call(t1, ed3[2])
    x4 = _dense3(x3, t1, q2, dinv23[2].reshape(N, 1), p[3]['Ws'][0],
                 p[3]['Ws'][1], p[3]['Ws'][2], vec(p[3]['b']),
                 vec(p[3]['gamma']), vec(p[3]['beta']))

    batch3 = batch.astype(jnp.int32).reshape(_GRID_TC, 1, _RB_TC)
    return _pool(x4, batch3)


# speedup vs baseline: 25.3767x; 1.0198x over previous
import jax
import jax.numpy as jnp
from jax import lax
from jax.experimental import pallas as pl
from jax.experimental.pallas import tpu as pltpu
from jax.experimental.pallas import tpu_sc as plsc

N = 10000
E = 320000
D = 128
G = 64
NC = 2
NS = 16
NW = NC * NS
CH = 112
NCHUNK = 90
EPT = NCHUNK * CH
EPAD = NW * EPT
RPT = 632
RPT_LAST = N - 15 * RPT
ACCN = 32768
DEG_PT = ACCN // NS
NLAY = 3
LANES = 16
NBUF = 3
NSLOT = 6
UNROLL = 6
NGRP = NCHUNK // UNROLL


def _zero_rows(buf, nrows, width):
    def body(r, c):
        for j in range(width // LANES):
            buf.at[r][pl.ds(j * LANES, LANES)] = jnp.zeros((LANES,),
                                                           jnp.float32)
        return c
    lax.fori_loop(0, nrows, body, 0)


def _deg_body(dst_hbm, ew_hbm, out_hbm, dst_v, ew_v, idx_v, zbuf, acc_s):
    cid = lax.axis_index("c")
    sid = lax.axis_index("s")
    wid = sid * NC + cid

    pltpu.sync_copy(dst_hbm.at[wid], dst_v)
    for l in range(NLAY):
        pltpu.sync_copy(ew_hbm.at[l, wid], ew_v.at[l])

    def zb(r, c):
        zbuf[pl.ds(r * LANES, LANES)] = jnp.zeros((LANES,), jnp.float32)
        return c
    lax.fori_loop(0, DEG_PT // LANES, zb, 0)
    pltpu.sync_copy(zbuf, acc_s.at[pl.ds(sid * DEG_PT, DEG_PT)])
    plsc.subcore_barrier()

    def chunk(g, c):
        for l in range(NLAY):
            for j in range(CH // LANES):
                sl = pl.ds(j * LANES, LANES)
                idx_v.at[l][sl] = dst_v.at[g][sl] + l * N
            pltpu.sync_copy(ew_v.at[l, g], acc_s.at[idx_v.at[l]], add=True)
        return c

    lax.fori_loop(0, NCHUNK, chunk, 0)
    plsc.subcore_barrier()

    pltpu.sync_copy(acc_s.at[pl.ds(sid * DEG_PT, DEG_PT)], zbuf)
    pltpu.sync_copy(zbuf, out_hbm.at[cid, 0, pl.ds(sid * DEG_PT, DEG_PT)])


_deg_call = pl.kernel(
    _deg_body,
    out_type=jax.ShapeDtypeStruct((NC, 1, ACCN), jnp.float32),
    mesh=plsc.VectorSubcoreMesh(
        core_axis_name="c", subcore_axis_name="s", num_cores=NC,
        num_subcores=NS),
    scratch_types=[
        pltpu.VMEM((NCHUNK, CH), jnp.int32),
        pltpu.VMEM((NLAY, NCHUNK, CH), jnp.float32),
        pltpu.VMEM((NLAY, CH), jnp.int32),
        pltpu.VMEM((DEG_PT,), jnp.float32),
        pltpu.VMEM_SHARED((ACCN,), jnp.float32),
    ],
    compiler_params=pltpu.CompilerParams(needs_layout_passes=False),
)


def _norm_body(src_hbm, dst_hbm, ew_hbm, dinv_hbm, out0, out1, out2,
               sd_v, ew_v, dinv_v, o_v):
    cid = lax.axis_index("c")
    sid = lax.axis_index("s")
    wid = sid * NC + cid

    pltpu.sync_copy(src_hbm.at[wid], sd_v.at[0])
    pltpu.sync_copy(dst_hbm.at[wid], sd_v.at[1])
    pltpu.sync_copy(dinv_hbm, dinv_v)

    def cp(g, c):
        for r in range(2):
            for j in range(CH // LANES):
                sl = pl.ds(j * LANES, LANES)
                o_v.at[g, r][sl] = sd_v.at[r, g][sl]
        return c

    lax.fori_loop(0, NCHUNK, cp, 0)

    for l, out_hbm in enumerate((out0, out1, out2)):
        pltpu.sync_copy(ew_hbm.at[l, wid], ew_v)
        l16 = jnp.full((LANES,), l, jnp.int32)

        def f(g, c):
            for j in range(CH // LANES):
                sl = pl.ds(j * LANES, LANES)
                s16 = sd_v.at[0, g][sl]
                d16 = sd_v.at[1, g][sl]
                w16 = ew_v.at[g][sl]
                dvs = plsc.load_gather(dinv_v, [l16, s16])
                dvd = plsc.load_gather(dinv_v, [l16, d16])
                o_v.at[g, 2][sl] = plsc.bitcast(-(dvs * w16 * dvd),
                                                jnp.int32)
            return c

        lax.fori_loop(0, NCHUNK, f, 0)
        pltpu.sync_copy(o_v, out_hbm.at[wid])


_norm_call = pl.kernel(
    _norm_body,
    out_type=tuple(
        jax.ShapeDtypeStruct((NW, NCHUNK, 3, CH), jnp.int32)
        for _ in range(NLAY)),
    mesh=plsc.VectorSubcoreMesh(
        core_axis_name="c", subcore_axis_name="s", num_cores=NC,
        num_subcores=NS),
    scratch_types=[
        pltpu.VMEM((2, NCHUNK, CH), jnp.int32),
        pltpu.VMEM((NCHUNK, CH), jnp.float32),
        pltpu.VMEM((NLAY, N), jnp.float32),
        pltpu.VMEM((NCHUNK, 3, CH), jnp.int32),
    ],
    compiler_params=pltpu.CompilerParams(needs_layout_passes=False),
)


def _lhat_body(v_hbm, ed_hbm, out_hbm, ering, rows, normc, acc_s,
               gsem, ssem, esem):
    cid = lax.axis_index("c")
    sid = lax.axis_index("s")
    wid = sid * NC + cid

    _zero_rows(rows.at[0], CH, D)
    base = sid * RPT

    @pl.when(sid < 15)
    def _():
        for k in range(RPT // CH):
            pltpu.sync_copy(rows.at[0], acc_s.at[pl.ds(base + k * CH, CH)])
        rem = RPT - (RPT // CH) * CH
        pltpu.sync_copy(rows.at[0, pl.ds(0, rem)],
                        acc_s.at[pl.ds(base + RPT - rem, rem)])

    @pl.when(sid == 15)
    def _():
        for k in range(RPT_LAST // CH):
            pltpu.sync_copy(rows.at[0], acc_s.at[pl.ds(base + k * CH, CH)])
        rem = RPT_LAST - (RPT_LAST // CH) * CH
        pltpu.sync_copy(rows.at[0, pl.ds(0, rem)],
                        acc_s.at[pl.ds(base + RPT_LAST - rem, rem)])

    plsc.subcore_barrier()

    def fetch_ed(c, slot):
        pltpu.async_copy(ed_hbm.at[wid, c], ering.at[slot], esem.at[slot])

    def wait_ed(slot):
        pltpu.make_async_copy(ed_hbm.at[wid, 0], ering.at[0],
                              esem.at[slot]).wait()

    def issue_gather(c, slot, b):
        pltpu.async_copy(v_hbm.at[ering.at[slot, 0]], rows.at[b], gsem.at[b])

    def wait_gather(b):
        pltpu.make_async_copy(v_hbm.at[pl.ds(0, CH)], rows.at[b],
                              gsem.at[b]).wait()

    def issue_scatter(slot, b):
        pltpu.async_copy(rows.at[b], acc_s.at[ering.at[slot, 1]],
                         ssem.at[b], add=True)

    def wait_scatter(b):
        pltpu.make_async_copy(rows.at[0], acc_s.at[pl.ds(0, CH)],
                              ssem.at[b]).wait()

    def scale(cdyn, slot, b):
        for j in range(CH // LANES):
            sl = pl.ds(j * LANES, LANES)
            normc[sl] = plsc.bitcast(ering.at[slot, 2][sl], jnp.float32)

        def srow(e, cc):
            ev = jnp.full((LANES,), e, jnp.int32)
            sc = plsc.load_gather(normc, [ev])
            for j in range(D // LANES):
                sl = pl.ds(j * LANES, LANES)
                rows.at[b, e][sl] = rows.at[b, e][sl] * sc
            return cc

        lax.fori_loop(0, CH, srow, 0, unroll=2)

    def step(c, k, first=False, pf=True, gi=True):
        b = k % NBUF
        wait_gather(b)
        scale(c, k % NSLOT, b)
        issue_scatter(k % NSLOT, b)
        if not first:
            wait_scatter((b + 2) % NBUF)
        if pf:
            fetch_ed(c + 4, (k + 4) % NSLOT)
        if gi:
            s2 = (k + 2) % NSLOT
            wait_ed(s2)
            issue_gather(c + 2, s2, (b + 2) % NBUF)

    for t in range(4):
        fetch_ed(t, t)
    for t in range(2):
        wait_ed(t)
        issue_gather(t, t, t)

    for k in range(UNROLL):
        step(k, k, first=(k == 0))

    def grp(g, c):
        c0 = g * UNROLL
        for k in range(UNROLL):
            step(c0 + k, k)
        return c

    lax.fori_loop(1, NGRP - 1, grp, 0)

    for k in range(UNROLL):
        c = NCHUNK - UNROLL + k
        step(c, k, pf=(c + 4 < NCHUNK), gi=(c + 2 < NCHUNK))

    wait_scatter((NCHUNK - 1) % NBUF)
    plsc.subcore_barrier()

    @pl.when(sid < 15)
    def _():
        pltpu.sync_copy(acc_s.at[pl.ds(base, RPT)],
                        out_hbm.at[cid, pl.ds(base, RPT)])

    @pl.when(sid == 15)
    def _():
        pltpu.sync_copy(acc_s.at[pl.ds(base, RPT_LAST)],
                        out_hbm.at[cid, pl.ds(base, RPT_LAST)])


_lhat_call = pl.kernel(
    _lhat_body,
    out_type=jax.ShapeDtypeStruct((NC, N, D), jnp.float32),
    mesh=plsc.VectorSubcoreMesh(
        core_axis_name="c", subcore_axis_name="s", num_cores=NC,
        num_subcores=NS),
    scratch_types=[
        pltpu.VMEM((NSLOT, 3, CH), jnp.int32),
        pltpu.VMEM((NBUF, CH, D), jnp.float32),
        pltpu.VMEM((CH,), jnp.float32),
        pltpu.VMEM_SHARED((N, D), jnp.float32),
        pltpu.SemaphoreType.DMA((NBUF,)),
        pltpu.SemaphoreType.DMA((NBUF,)),
        pltpu.SemaphoreType.DMA((NSLOT,)),
    ],
    compiler_params=pltpu.CompilerParams(needs_layout_passes=False),
)


def _ew_body(we_ref, ea_ref, o_ref):
    z = lax.dot_general(we_ref[...], ea_ref[...], (((1,), (1,)), ((), ())),
                        preferred_element_type=jnp.float32)
    o_ref[...] = 1.0 / (1.0 + jnp.exp(-z))


def _ew_sigmoid(we_t, edge_attr):
    eb = 16000
    return pl.pallas_call(
        _ew_body,
        grid=(E // eb,),
        in_specs=[
            pl.BlockSpec((NLAY, 16), lambda i: (0, 0)),
            pl.BlockSpec((eb, 16), lambda i: (i, 0)),
        ],
        out_specs=pl.BlockSpec((NLAY, eb), lambda i: (0, i)),
        out_shape=jax.ShapeDtypeStruct((NLAY, E), jnp.float32),
    )(we_t, edge_attr)


def _dinv_body(degp_ref, dinv_ref, dinv2_ref):
    dtot = degp_ref[0] + degp_ref[1] + 1.0
    dinv2 = 1.0 / dtot
    dinv_ref[...] = lax.rsqrt(dtot)
    dinv2_ref[...] = dinv2


def _dinv_fn(degp):
    return pl.pallas_call(
        _dinv_body,
        out_shape=(
            jax.ShapeDtypeStruct((NLAY, N), jnp.float32),
            jax.ShapeDtypeStruct((NLAY, N), jnp.float32),
        ),
    )(degp)


def _ln_relu(h, g, b):
    mu = jnp.mean(h, axis=-1, keepdims=True)
    var = jnp.mean((h - mu) * (h - mu), axis=-1, keepdims=True)
    hn = (h - mu) * lax.rsqrt(var + 1e-5) * g + b
    return jnp.maximum(hn, 0.0)


_RB_TC = 2000
_GRID_TC = N // _RB_TC


def _rows_spec():
    return pl.BlockSpec((_RB_TC, D), lambda i: (i, 0))


def _part_spec():
    return pl.BlockSpec((NC, _RB_TC, D), lambda i: (0, i, 0))


def _vec_spec():
    return pl.BlockSpec((1, D), lambda i: (0, 0))


def _col_spec():
    return pl.BlockSpec((_RB_TC, 1), lambda i: (i, 0))


def _w_spec():
    return pl.BlockSpec((D, D), lambda i: (0, 0))


def _dense1_body(x_ref, w0_ref, b_ref, g_ref, be_ref, o_ref):
    h = jnp.dot(x_ref[...], w0_ref[...], preferred_element_type=jnp.float32)
    o_ref[...] = _ln_relu(h + b_ref[...], g_ref[...], be_ref[...])


def _dense1(x, w0, b, g, be):
    return pl.pallas_call(
        _dense1_body,
        grid=(_GRID_TC,),
        in_specs=[_rows_spec(), _w_spec(), _vec_spec(), _vec_spec(),
                  _vec_spec()],
        out_specs=_rows_spec(),
        out_shape=jax.ShapeDtypeStruct((N, D), jnp.float32),
    )(x, w0, b, g, be)


def _dense2_body(x_ref, p_ref, d2_ref, w0_ref, w1_ref, b_ref, g_ref, be_ref,
                 o_ref):
    x = x_ref[...]
    t1 = p_ref[0] + p_ref[1] - d2_ref[...] * x
    h = (jnp.dot(x, w0_ref[...], preferred_element_type=jnp.float32)
         + jnp.dot(t1, w1_ref[...], preferred_element_type=jnp.float32)
         + b_ref[...])
    o_ref[...] = _ln_relu(h, g_ref[...], be_ref[...])


def _dense2(x, p, d2, w0, w1, b, g, be):
    return pl.pallas_call(
        _dense2_body,
        grid=(_GRID_TC,),
        in_specs=[_rows_spec(), _part_spec(), _col_spec(), _w_spec(),
                  _w_spec(), _vec_spec(), _vec_spec(), _vec_spec()],
        out_specs=_rows_spec(),
        out_shape=jax.ShapeDtypeStruct((N, D), jnp.float32),
    )(x, p, d2, w0, w1, b, g, be)


def _comb_body(x_ref, p_ref, d2_ref, o_ref):
    o_ref[...] = p_ref[0] + p_ref[1] - d2_ref[...] * x_ref[...]


def _comb(x, p, d2):
    return pl.pallas_call(
        _comb_body,
        grid=(_GRID_TC,),
        in_specs=[_rows_spec(), _part_spec(), _col_spec()],
        out_specs=_rows_spec(),
        out_shape=jax.ShapeDtypeStruct((N, D), jnp.float32),
    )(x, p, d2)


def _dense3_body(x_ref, t1_ref, q_ref, d2_ref, w0_ref, w1_ref, w2_ref, b_ref,
                 g_ref, be_ref, o_ref):
    x = x_ref[...]
    t1 = t1_ref[...]
    t2 = 2.0 * (q_ref[0] + q_ref[1] - d2_ref[...] * t1) - x
    h = (jnp.dot(x, w0_ref[...], preferred_element_type=jnp.float32)
         + jnp.dot(t1, w1_ref[...], preferred_element_type=jnp.float32)
         + jnp.dot(t2, w2_ref[...], preferred_element_type=jnp.float32)
         + b_ref[...])
    o_ref[...] = _ln_relu(h, g_ref[...], be_ref[...])


def _dense3(x, t1, q, d2, w0, w1, w2, b, g, be):
    return pl.pallas_call(
        _dense3_body,
        grid=(_GRID_TC,),
        in_specs=[_rows_spec(), _rows_spec(), _part_spec(), _col_spec(),
                  _w_spec(), _w_spec(), _w_spec(), _vec_spec(), _vec_spec(),
                  _vec_spec()],
        out_specs=_rows_spec(),
        out_shape=jax.ShapeDtypeStruct((N, D), jnp.float32),
    )(x, t1, q, d2, w0, w1, w2, b, g, be)


def _dense3_pool_body(x_ref, t1_ref, q_ref, d2_ref, w0_ref, w1_ref, w2_ref,
                      b_ref, g_ref, be_ref, bt_ref, o_ref, sum_s, cnt_s):
    i = pl.program_id(0)

    @pl.when(i == 0)
    def _():
        sum_s[...] = jnp.zeros((G, D), jnp.float32)
        cnt_s[...] = jnp.zeros((G, D), jnp.float32)

    x = x_ref[...]
    t1 = t1_ref[...]
    t2 = 2.0 * (q_ref[0] + q_ref[1] - d2_ref[...] * t1) - x
    h = (jnp.dot(x, w0_ref[...], preferred_element_type=jnp.float32)
         + jnp.dot(t1, w1_ref[...], preferred_element_type=jnp.float32)
         + jnp.dot(t2, w2_ref[...], preferred_element_type=jnp.float32)
         + b_ref[...])
    x4 = _ln_relu(h, g_ref[...], be_ref[...])

    bb = bt_ref[0, 0, :]
    seg = lax.broadcasted_iota(jnp.int32, (G, _RB_TC), 0)
    oh = (seg == bb[None, :]).astype(jnp.float32)
    sum_s[...] += jnp.dot(oh, x4, preferred_element_type=jnp.float32)
    cnt_s[...] += jnp.dot(oh, jnp.ones_like(x4),
                          preferred_element_type=jnp.float32)

    @pl.when(i == pl.num_programs(0) - 1)
    def _():
        o_ref[...] = sum_s[...] / jnp.maximum(cnt_s[...], 1.0)


def _dense3_pool(x, t1, q, d2, w0, w1, w2, b, g, be, batch3):
    return pl.pallas_call(
        _dense3_pool_body,
        grid=(_GRID_TC,),
        in_specs=[_rows_spec(), _rows_spec(), _part_spec(), _col_spec(),
                  _w_spec(), _w_spec(), _w_spec(), _vec_spec(), _vec_spec(),
                  _vec_spec(),
                  pl.BlockSpec((1, 1, _RB_TC), lambda i: (i, 0, 0))],
        out_specs=pl.BlockSpec((G, D), lambda i: (0, 0)),
        out_shape=jax.ShapeDtypeStruct((G, D), jnp.float32),
        scratch_shapes=[
            pltpu.VMEM((G, D), jnp.float32),
            pltpu.VMEM((G, D), jnp.float32),
        ],
    )(x, t1, q, d2, w0, w1, w2, b, g, be, batch3)


def _pool_body(x_ref, b_ref, o_ref, sum_s, cnt_s):
    i = pl.program_id(0)

    @pl.when(i == 0)
    def _():
        sum_s[...] = jnp.zeros((G, D), jnp.float32)
        cnt_s[...] = jnp.zeros((G, D), jnp.float32)

    bb = b_ref[0, 0, :]
    seg = lax.broadcasted_iota(jnp.int32, (G, _RB_TC), 0)
    oh = (seg == bb[None, :]).astype(jnp.float32)
    x = x_ref[...]
    sum_s[...] += jnp.dot(oh, x, preferred_element_type=jnp.float32)
    cnt_s[...] += jnp.dot(oh, jnp.ones_like(x),
                          preferred_element_type=jnp.float32)

    @pl.when(i == pl.num_programs(0) - 1)
    def _():
        o_ref[...] = sum_s[...] / jnp.maximum(cnt_s[...], 1.0)


def _pool(x, batch3):
    return pl.pallas_call(
        _pool_body,
        grid=(_GRID_TC,),
        in_specs=[
            _rows_spec(),
            pl.BlockSpec((1, 1, _RB_TC), lambda i: (i, 0, 0)),
        ],
        out_specs=pl.BlockSpec((G, D), lambda i: (0, 0)),
        out_shape=jax.ShapeDtypeStruct((G, D), jnp.float32),
        scratch_shapes=[
            pltpu.VMEM((G, D), jnp.float32),
            pltpu.VMEM((G, D), jnp.float32),
        ],
    )(x, batch3)


def kernel(x, edge_attr, params, edge_index, batch):
    src = edge_index[0].astype(jnp.int32)
    dst = edge_index[1].astype(jnp.int32)

    npad = EPAD - E
    padidx = jnp.arange(npad, dtype=jnp.int32)
    src_p = jnp.concatenate([src, padidx]).reshape(NW, NCHUNK, CH)
    dst_p = jnp.concatenate([dst, padidx]).reshape(NW, NCHUNK, CH)

    we_t = jnp.stack([params['layer%d' % i]['we'] for i in (1, 2, 3)])
    ew = _ew_sigmoid(we_t, edge_attr)
    ew3 = jnp.pad(ew, ((0, 0), (0, npad))).reshape(NLAY, NW, NCHUNK, CH)

    degp = _deg_call(dst_p, ew3)
    degp = degp.reshape(NC, ACCN)[:, :NLAY * N].reshape(NC, NLAY, N)
    dinv3, dinv23 = _dinv_fn(degp)
    ed0, ed1, ed2 = _norm_call(src_p, dst_p, ew3, dinv3)

    p = [params['layer%d' % i] for i in range(4)]

    def vec(a):
        return a.reshape(1, D)

    x1 = _dense1(x, p[0]['Ws'][0], vec(p[0]['b']), vec(p[0]['gamma']),
                 vec(p[0]['beta']))

    p1 = _lhat_call(x1, ed0)
    x2 = _dense2(x1, p1, dinv23[0].reshape(N, 1), p[1]['Ws'][0],
                 p[1]['Ws'][1], vec(p[1]['b']), vec(p[1]['gamma']),
                 vec(p[1]['beta']))

    p2 = _lhat_call(x2, ed1)
    x3 = _dense2(x2, p2, dinv23[1].reshape(N, 1), p[2]['Ws'][0],
                 p[2]['Ws'][1], vec(p[2]['b']), vec(p[2]['gamma']),
                 vec(p[2]['beta']))

    q1 = _lhat_call(x3, ed2)
    t1 = _comb(x3, q1, dinv23[2].reshape(N, 1))
    q2 = _lhat_call(t1, ed2)
    batch3 = batch.astype(jnp.int32).reshape(_GRID_TC, 1, _RB_TC)
    return _dense3_pool(x3, t1, q2, dinv23[2].reshape(N, 1), p[3]['Ws'][0],
                        p[3]['Ws'][1], p[3]['Ws'][2], vec(p[3]['b']),
                        vec(p[3]['gamma']), vec(p[3]['beta']), batch3)
